# baseline (speedup 1.0000x reference)
import functools

import jax
import jax.numpy as jnp
from jax import lax
from jax.experimental import pallas as pl
from jax.experimental.pallas import tpu as pltpu
from jax.experimental.pallas import tpu_sc as plsc

N = 10000
E = 320000
EMBED = 128
HEADS = 8
HEAD_DIM = EMBED // HEADS
XW = 16
QW = 256

NC = 2
NS = 16
NW = NC * NS
EPW = E // NW
CH = 80
NCHUNK = EPW // CH
PAIRS = NCHUNK // 2
GCH = 128
GFULL = EPW // GCH
GPAIRS = GFULL // 2
GTAIL = EPW - GFULL * GCH
GPAD = GCH - GTAIL
GNCH = GFULL + 1
NPS = N // NS
STRIPE = 624
TAIL = N - NS * STRIPE

BN = 1000
BE = 6400

_F32 = jnp.float32


def _ln(h, g, b):
    m = jnp.mean(h, axis=-1, keepdims=True)
    v = jnp.mean((h - m) ** 2, axis=-1, keepdims=True)
    return (h - m) / jnp.sqrt(v + 1e-5) * g + b


def _dot(a, b):
    return jnp.dot(a, b, preferred_element_type=_F32)


def _node_body(x_ref, rm_ref, bm_ref, bos_ref,
               w1, b1, g1, bb1, w2, b2, g2, bb2, w3, b3, g3, bb3, wq, bq,
               center_ref, qs_ref):
    x0 = x_ref[:, 0:1]
    x1 = x_ref[:, 1:2]
    r0 = rm_ref[:, 0:1]
    r1 = rm_ref[:, 1:2]
    r2 = rm_ref[:, 2:3]
    r3 = rm_ref[:, 3:4]
    xr0 = x0 * r0 + x1 * r2
    xr1 = x0 * r1 + x1 * r3
    h = xr0 * w1[0:1, :] + xr1 * w1[1:2, :] + b1[...]
    h = jax.nn.relu(_ln(h, g1[...], bb1[...]))
    h = _dot(h, w2[...]) + b2[...]
    h = jax.nn.relu(_ln(h, g2[...], bb2[...]))
    h = _dot(h, w3[...]) + b3[...]
    c = _ln(h, g3[...], bb3[...])
    m = bm_ref[...]
    c = c + m * (bos_ref[...] - c)
    center_ref[...] = c
    qs_ref[...] = (_dot(c, wq[...]) + bq[...]) * 0.25


def _node_stage(x, rm4, bm, bos_row, p):
    full = lambda shp: pl.BlockSpec(shp, lambda i: tuple(0 for _ in shp))
    row = lambda d: pl.BlockSpec((BN, d), lambda i: (i, 0))
    w = lambda k: p[k]
    vec = lambda k: p[k].reshape(1, -1)
    ins = [x, rm4, bm, bos_row,
           w('ce_w1'), vec('ce_b1'), vec('ce_g1'), vec('ce_bb1'),
           w('ce_w2'), vec('ce_b2'), vec('ce_g2'), vec('ce_bb2'),
           w('ce_w3'), vec('ce_b3'), vec('ce_g3'), vec('ce_bb3'),
           w('wq'), vec('bq')]
    in_specs = [row(2), row(4), row(1), full((1, EMBED))] + [
        full(a.shape) for a in ins[4:]]
    return pl.pallas_call(
        _node_body,
        grid=(N // BN,),
        in_specs=in_specs,
        out_specs=[row(EMBED), row(EMBED)],
        out_shape=[jax.ShapeDtypeStruct((N, EMBED), _F32),
                   jax.ShapeDtypeStruct((N, EMBED), _F32)],
    )(*ins)


def _sc_gather_x_body(x_hbm, src2_hbm, xg_hbm,
                      src_v, xrow_a, xrow_b, sem_a, sem_b):
    wid = lax.axis_index("s") * NC + lax.axis_index("c")
    base = wid * EPW
    pltpu.sync_copy(src2_hbm.at[wid], src_v)

    def pair(j, carry):
        c0 = 2 * j
        c1 = c0 + 1
        a = pltpu.async_copy(x_hbm.at[src_v.at[c0]], xrow_a, sem_a)
        b = pltpu.async_copy(x_hbm.at[src_v.at[c1]], xrow_b, sem_b)
        a.wait()
        pltpu.sync_copy(xrow_a, xg_hbm.at[pl.ds(base + c0 * GCH, GCH)])
        b.wait()
        pltpu.sync_copy(xrow_b, xg_hbm.at[pl.ds(base + c1 * GCH, GCH)])
        return carry

    lax.fori_loop(0, GPAIRS, pair, 0)
    a = pltpu.async_copy(x_hbm.at[src_v.at[GFULL]], xrow_a, sem_a)
    a.wait()
    pltpu.sync_copy(xrow_a.at[pl.ds(0, GTAIL)],
                    xg_hbm.at[pl.ds(base + GFULL * GCH, GTAIL)])


@functools.lru_cache(maxsize=None)
def _sc_gather_x_call():
    return pl.kernel(
        _sc_gather_x_body,
        out_type=jax.ShapeDtypeStruct((E, XW), _F32),
        mesh=plsc.VectorSubcoreMesh(core_axis_name="c", subcore_axis_name="s",
                                    num_cores=NC, num_subcores=NS),
        scratch_types=[pltpu.VMEM((GNCH, GCH), jnp.int32),
                       pltpu.VMEM((GCH, XW), _F32),
                       pltpu.VMEM((GCH, XW), _F32),
                       pltpu.SemaphoreType.DMA,
                       pltpu.SemaphoreType.DMA],
        compiler_params=pltpu.CompilerParams(use_tc_tiling_on_sc=False),
    )


def _sc_gather_q_body(q_hbm, dst2_hbm, qg_hbm,
                      dst_v, row_a, row_b, sem_a, sem_b):
    wid = lax.axis_index("s") * NC + lax.axis_index("c")
    base = wid * EPW
    pltpu.sync_copy(dst2_hbm.at[wid], dst_v)

    def pair(j, carry):
        c0 = 2 * j
        c1 = c0 + 1
        a = pltpu.async_copy(q_hbm.at[dst_v.at[c0]], row_a, sem_a)
        b = pltpu.async_copy(q_hbm.at[dst_v.at[c1]], row_b, sem_b)
        a.wait()
        pltpu.sync_copy(row_a, qg_hbm.at[pl.ds(base + c0 * GCH, GCH)])
        b.wait()
        pltpu.sync_copy(row_b, qg_hbm.at[pl.ds(base + c1 * GCH, GCH)])
        return carry

    lax.fori_loop(0, GPAIRS, pair, 0)
    a = pltpu.async_copy(q_hbm.at[dst_v.at[GFULL]], row_a, sem_a)
    a.wait()
    pltpu.sync_copy(row_a.at[pl.ds(0, GTAIL)],
                    qg_hbm.at[pl.ds(base + GFULL * GCH, GTAIL)])


@functools.lru_cache(maxsize=None)
def _sc_gather_q_call():
    return pl.kernel(
        _sc_gather_q_body,
        out_type=jax.ShapeDtypeStruct((E, QW), _F32),
        mesh=plsc.VectorSubcoreMesh(core_axis_name="c", subcore_axis_name="s",
                                    num_cores=NC, num_subcores=NS),
        scratch_types=[pltpu.VMEM((GNCH, GCH), jnp.int32),
                       pltpu.VMEM((GCH, QW), _F32),
                       pltpu.VMEM((GCH, QW), _F32),
                       pltpu.SemaphoreType.DMA,
                       pltpu.SemaphoreType.DMA],
    )


def _edge_body(xg_ref, ea_ref, qrm_ref,
               n0w1, n0b1, n0g1, n0bb1,
               n1w1, n1b1, n1g1, n1bb1, w02, b02,
               ag1, ab1, aw, ab,
               wkv, bkv, s_ref, st_ref,
               ev_ref, ex_ref):
    qg = qrm_ref[:, 0:EMBED]
    r0 = qrm_ref[:, EMBED:EMBED + 1]
    r1 = qrm_ref[:, EMBED + 1:EMBED + 2]
    r2 = qrm_ref[:, EMBED + 2:EMBED + 3]
    r3 = qrm_ref[:, EMBED + 3:EMBED + 4]
    x0 = xg_ref[:, 0:1]
    x1 = xg_ref[:, 1:2]
    e0 = ea_ref[:, 0:1]
    e1 = ea_ref[:, 1:2]
    nx0 = x0 * r0 + x1 * r2
    nx1 = x0 * r1 + x1 * r3
    ne0 = e0 * r0 + e1 * r2
    ne1 = e0 * r1 + e1 * r3
    h0 = nx0 * n0w1[0:1, :] + nx1 * n0w1[1:2, :] + n0b1[...]
    h0 = jax.nn.relu(_ln(h0, n0g1[...], n0bb1[...]))
    h1 = ne0 * n1w1[0:1, :] + ne1 * n1w1[1:2, :] + n1b1[...]
    h1 = jax.nn.relu(_ln(h1, n1g1[...], n1bb1[...]))
    s = _dot(jnp.concatenate([h0, h1], axis=1), w02[...]) + b02[...]
    nbr = jax.nn.relu(_ln(s, ag1[...], ab1[...]))
    h = _dot(nbr, aw[...]) + ab[...]
    m = jnp.mean(h, axis=-1, keepdims=True)
    v2 = jnp.mean((h - m) ** 2, axis=-1, keepdims=True)
    nbr = (h - m) / jnp.sqrt(v2 + 1e-5)
    kv = _dot(nbr, wkv[...]) + bkv[...]
    k = kv[:, 0:EMBED]
    v = kv[:, EMBED:2 * EMBED]
    alpha = _dot(qg * k, s_ref[...])
    ex = jnp.exp(alpha)
    ev_ref[...] = v * _dot(ex, st_ref[...])
    ex_ref[...] = ex


def _edge_stage(xg, ea, qrm, smat, stmat, p):
    full = lambda shp: pl.BlockSpec(shp, lambda i: tuple(0 for _ in shp))
    row = lambda d: pl.BlockSpec((BE, d), lambda i: (i, 0))
    w = lambda k: p[k]
    vec = lambda k: p[k].reshape(1, -1)
    w02 = jnp.concatenate([p['n0_w2'], p['n1_w2']], axis=0)
    b02 = (p['n0_b2'] + p['n1_b2']).reshape(1, -1)
    wkv0 = jnp.concatenate([p['wk'], p['wv']], axis=1)
    bkv0 = jnp.concatenate([p['bk'], p['bv']])
    wkv = p['agg_g2'][:, None] * wkv0
    bkv = (p['agg_b2'] @ wkv0 + bkv0).reshape(1, -1)
    ins = [xg, ea, qrm,
           w('n0_w1'), vec('n0_b1'), vec('n0_g1'), vec('n0_bb1'),
           w('n1_w1'), vec('n1_b1'), vec('n1_g1'), vec('n1_bb1'),
           w02, b02,
           vec('agg_g1'), vec('agg_b1'), w('agg_w'), vec('agg_b'),
           wkv, bkv, smat, stmat]
    in_specs = [row(XW), row(2), row(QW)] + [
        full(a.shape) for a in ins[3:]]
    return pl.pallas_call(
        _edge_body,
        grid=(E // BE,),
        in_specs=in_specs,
        out_specs=[row(EMBED), row(HEADS)],
        out_shape=[jax.ShapeDtypeStruct((E, EMBED), _F32),
                   jax.ShapeDtypeStruct((E, HEADS), _F32)],
    )(*ins)


def _sc_scatter_ev_body(ev_hbm, dst2_hbm, z128_hbm, msgp_hbm,
                        dst_v, ev_a, ev_b, sem_a, sem_b, msg_sh):
    cid = lax.axis_index("c")
    sid = lax.axis_index("s")
    wid = sid * NC + cid
    base = wid * EPW
    pltpu.sync_copy(dst2_hbm.at[wid], dst_v)
    pltpu.sync_copy(z128_hbm.at[pl.ds(sid * STRIPE, STRIPE)],
                    msg_sh.at[pl.ds(sid * STRIPE, STRIPE)])

    @pl.when(sid == 0)
    def _():
        pltpu.sync_copy(z128_hbm.at[pl.ds(NS * STRIPE, TAIL)],
                        msg_sh.at[pl.ds(NS * STRIPE, TAIL)])

    plsc.subcore_barrier()

    def pair(j, carry):
        c0 = 2 * j
        c1 = c0 + 1
        a = pltpu.async_copy(ev_hbm.at[pl.ds(base + c0 * CH, CH)], ev_a, sem_a)
        b = pltpu.async_copy(ev_hbm.at[pl.ds(base + c1 * CH, CH)], ev_b, sem_b)
        a.wait()
        pltpu.sync_copy(ev_a, msg_sh.at[dst_v.at[c0]], add=True)
        b.wait()
        pltpu.sync_copy(ev_b, msg_sh.at[dst_v.at[c1]], add=True)
        return carry

    lax.fori_loop(0, PAIRS, pair, 0)
    c = NCHUNK - 1
    a = pltpu.async_copy(ev_hbm.at[pl.ds(base + c * CH, CH)], ev_a, sem_a)
    a.wait()
    pltpu.sync_copy(ev_a, msg_sh.at[dst_v.at[c]], add=True)
    plsc.subcore_barrier()
    pltpu.sync_copy(msg_sh.at[pl.ds(sid * STRIPE, STRIPE)],
                    msgp_hbm.at[cid, pl.ds(sid * STRIPE, STRIPE)])

    @pl.when(sid == 0)
    def _():
        pltpu.sync_copy(msg_sh.at[pl.ds(NS * STRIPE, TAIL)],
                        msgp_hbm.at[cid, pl.ds(NS * STRIPE, TAIL)])


@functools.lru_cache(maxsize=None)
def _sc_scatter_ev_call():
    return pl.kernel(
        _sc_scatter_ev_body,
        out_type=jax.ShapeDtypeStruct((NC, N, EMBED), _F32),
        mesh=plsc.VectorSubcoreMesh(core_axis_name="c", subcore_axis_name="s",
                                    num_cores=NC, num_subcores=NS),
        scratch_types=[pltpu.VMEM((NCHUNK, CH), jnp.int32),
                       pltpu.VMEM((CH, EMBED), _F32),
                       pltpu.VMEM((CH, EMBED), _F32),
                       pltpu.SemaphoreType.DMA,
                       pltpu.SemaphoreType.DMA,
                       pltpu.VMEM_SHARED((N, EMBED), _F32)],
    )


def _sc_scatter_ex_body(ex_hbm, dst2_hbm, z8_hbm, denp_hbm,
                        dst_v, ex_a, ex_b, sem_a, sem_b, den_sh):
    cid = lax.axis_index("c")
    sid = lax.axis_index("s")
    wid = sid * NC + cid
    base = wid * EPW
    pltpu.sync_copy(dst2_hbm.at[wid], dst_v)
    pltpu.sync_copy(z8_hbm.at[pl.ds(sid * NPS, NPS)],
                    den_sh.at[pl.ds(sid * NPS, NPS)])
    plsc.subcore_barrier()

    def pair(j, carry):
        c0 = 2 * j
        c1 = c0 + 1
        a = pltpu.async_copy(ex_hbm.at[pl.ds(base + c0 * CH, CH)], ex_a, sem_a)
        b = pltpu.async_copy(ex_hbm.at[pl.ds(base + c1 * CH, CH)], ex_b, sem_b)
        a.wait()
        pltpu.sync_copy(ex_a, den_sh.at[dst_v.at[c0]], add=True)
        b.wait()
        pltpu.sync_copy(ex_b, den_sh.at[dst_v.at[c1]], add=True)
        return carry

    lax.fori_loop(0, PAIRS, pair, 0)
    c = NCHUNK - 1
    a = pltpu.async_copy(ex_hbm.at[pl.ds(base + c * CH, CH)], ex_a, sem_a)
    a.wait()
    pltpu.sync_copy(ex_a, den_sh.at[dst_v.at[c]], add=True)
    plsc.subcore_barrier()
    pltpu.sync_copy(den_sh.at[pl.ds(sid * NPS, NPS)],
                    denp_hbm.at[cid, pl.ds(sid * NPS, NPS)])


@functools.lru_cache(maxsize=None)
def _sc_scatter_ex_call():
    return pl.kernel(
        _sc_scatter_ex_body,
        out_type=jax.ShapeDtypeStruct((NC, N, HEADS), _F32),
        mesh=plsc.VectorSubcoreMesh(core_axis_name="c", subcore_axis_name="s",
                                    num_cores=NC, num_subcores=NS),
        scratch_types=[pltpu.VMEM((NCHUNK, CH), jnp.int32),
                       pltpu.VMEM((CH, HEADS), _F32),
                       pltpu.VMEM((CH, HEADS), _F32),
                       pltpu.SemaphoreType.DMA,
                       pltpu.SemaphoreType.DMA,
                       pltpu.VMEM_SHARED((N, HEADS), _F32)],
        compiler_params=pltpu.CompilerParams(use_tc_tiling_on_sc=False),
    )


def _fuse_body(msgp_ref, denp_ref, c_ref, st_ref,
               wih, bih, whh, bhh, wself, bself, wout, bout,
               ln1g, ln1b, mw1, mb1, mw2, mb2, ln2g, ln2b,
               out_ref):
    den = denp_ref[0] + denp_ref[1]
    msg = (msgp_ref[0] + msgp_ref[1]) / (_dot(den, st_ref[...]) + 1e-16)
    c = c_ref[...]
    gate = jax.nn.sigmoid(_dot(msg, wih[...]) + bih[...]
                          + _dot(c, whh[...]) + bhh[...])
    o = msg + gate * (_dot(c, wself[...]) + bself[...] - msg)
    o = _dot(o, wout[...]) + bout[...]
    c1 = _ln(c + o, ln1g[...], ln1b[...])
    ff = _dot(jax.nn.relu(_dot(c1, mw1[...]) + mb1[...]), mw2[...]) + mb2[...]
    out_ref[...] = _ln(c1 + ff, ln2g[...], ln2b[...])


def _fuse_stage(msgp, denp, center, stmat, p):
    full = lambda shp: pl.BlockSpec(shp, lambda i: tuple(0 for _ in shp))
    w = lambda k: p[k]
    vec = lambda k: p[k].reshape(1, -1)
    ins = [msgp, denp, center, stmat,
           w('wih'), vec('bih'), w('whh'), vec('bhh'),
           w('wself'), vec('bself'), w('wout'), vec('bout'),
           vec('ln1_g'), vec('ln1_b'),
           w('mlp_w1'), vec('mlp_b1'), w('mlp_w2'), vec('mlp_b2'),
           vec('ln2_g'), vec('ln2_b')]
    in_specs = [pl.BlockSpec((NC, BN, EMBED), lambda i: (0, i, 0)),
                pl.BlockSpec((NC, BN, HEADS), lambda i: (0, i, 0)),
                pl.BlockSpec((BN, EMBED), lambda i: (i, 0))] + [
        full(a.shape) for a in ins[3:]]
    return pl.pallas_call(
        _fuse_body,
        grid=(N // BN,),
        in_specs=in_specs,
        out_specs=pl.BlockSpec((BN, EMBED), lambda i: (i, 0)),
        out_shape=jax.ShapeDtypeStruct((N, EMBED), _F32),
    )(*ins)


def kernel(x, edge_index, edge_attr, bos_mask, t, rotate_mat, params):
    p = params
    dst = edge_index[0].astype(jnp.int32)
    src = edge_index[1].astype(jnp.int32)
    rm4 = rotate_mat.reshape(N, 4)
    bm = bos_mask.astype(_F32).reshape(N, 1)
    bos_row = lax.dynamic_index_in_dim(p['bos'], t, axis=0, keepdims=True)
    jj = jnp.arange(EMBED)[:, None] // HEAD_DIM
    smat = (jj == jnp.arange(HEADS)[None, :]).astype(_F32)
    stmat = smat.T

    center, qs = _node_stage(x, rm4, bm, bos_row, p)
    xpad = jnp.concatenate([x, jnp.zeros((N, XW - 2), _F32)], axis=1)
    qrm_tab = jnp.concatenate([qs, rm4, jnp.zeros((N, QW - EMBED - 4), _F32)],
                              axis=1)
    dst2 = dst.reshape(NW, NCHUNK, CH)
    srcg = jnp.pad(src.reshape(NW, EPW), ((0, 0), (0, GPAD))
                   ).reshape(NW, GNCH, GCH)
    dstg = jnp.pad(dst.reshape(NW, EPW), ((0, 0), (0, GPAD))
                   ).reshape(NW, GNCH, GCH)
    xg = _sc_gather_x_call()(xpad, srcg)
    qrm = _sc_gather_q_call()(qrm_tab, dstg)
    ev, ex = _edge_stage(xg, edge_attr, qrm, smat, stmat, p)
    z128 = jnp.zeros((N, EMBED), _F32)
    z8 = jnp.zeros((N, HEADS), _F32)
    msgp = _sc_scatter_ev_call()(ev, dst2, z128)
    denp = _sc_scatter_ex_call()(ex, dst2, z8)
    return _fuse_stage(msgp, denp, center, stmat, p)


# speedup vs baseline: 1.0846x; 1.0846x over previous
import functools

import jax
import jax.numpy as jnp
from jax import lax
from jax.experimental import pallas as pl
from jax.experimental.pallas import tpu as pltpu
from jax.experimental.pallas import tpu_sc as plsc

N = 10000
E = 320000
EMBED = 128
HEADS = 8
HEAD_DIM = EMBED // HEADS
XW = 16
QW = 256

NC = 2
NS = 16
NW = NC * NS
EPW = E // NW
CH = 80
NCHUNK = EPW // CH
PAIRS = NCHUNK // 2
NPS = N // NS
STRIPE = 624
TAIL = N - NS * STRIPE

BN = 1000
BE = 6400

_F32 = jnp.float32


def _ln(h, g, b):
    m = jnp.mean(h, axis=-1, keepdims=True)
    v = jnp.mean((h - m) ** 2, axis=-1, keepdims=True)
    return (h - m) / jnp.sqrt(v + 1e-5) * g + b


def _dot(a, b):
    return jnp.dot(a, b, preferred_element_type=_F32)


def _node_body(x_ref, rm_ref, bm_ref, bos_ref,
               w1, b1, g1, bb1, w2, b2, g2, bb2, w3, b3, g3, bb3, wq, bq,
               center_ref, qs_ref):
    x0 = x_ref[:, 0:1]
    x1 = x_ref[:, 1:2]
    r0 = rm_ref[:, 0:1]
    r1 = rm_ref[:, 1:2]
    r2 = rm_ref[:, 2:3]
    r3 = rm_ref[:, 3:4]
    xr0 = x0 * r0 + x1 * r2
    xr1 = x0 * r1 + x1 * r3
    h = xr0 * w1[0:1, :] + xr1 * w1[1:2, :] + b1[...]
    h = jax.nn.relu(_ln(h, g1[...], bb1[...]))
    h = _dot(h, w2[...]) + b2[...]
    h = jax.nn.relu(_ln(h, g2[...], bb2[...]))
    h = _dot(h, w3[...]) + b3[...]
    c = _ln(h, g3[...], bb3[...])
    m = bm_ref[...]
    c = c + m * (bos_ref[...] - c)
    center_ref[...] = c
    qs_ref[...] = (_dot(c, wq[...]) + bq[...]) * 0.25


def _node_stage(x, rm4, bm, bos_row, p):
    full = lambda shp: pl.BlockSpec(shp, lambda i: tuple(0 for _ in shp))
    row = lambda d: pl.BlockSpec((BN, d), lambda i: (i, 0))
    w = lambda k: p[k]
    vec = lambda k: p[k].reshape(1, -1)
    ins = [x, rm4, bm, bos_row,
           w('ce_w1'), vec('ce_b1'), vec('ce_g1'), vec('ce_bb1'),
           w('ce_w2'), vec('ce_b2'), vec('ce_g2'), vec('ce_bb2'),
           w('ce_w3'), vec('ce_b3'), vec('ce_g3'), vec('ce_bb3'),
           w('wq'), vec('bq')]
    in_specs = [row(2), row(4), row(1), full((1, EMBED))] + [
        full(a.shape) for a in ins[4:]]
    return pl.pallas_call(
        _node_body,
        grid=(N // BN,),
        in_specs=in_specs,
        out_specs=[row(EMBED), row(EMBED)],
        out_shape=[jax.ShapeDtypeStruct((N, EMBED), _F32),
                   jax.ShapeDtypeStruct((N, EMBED), _F32)],
    )(*ins)


def _sc_gather_x_body(x_hbm, src2_hbm, xg_hbm,
                      src_v, xrow_a, xrow_b, sem_a, sem_b):
    wid = lax.axis_index("s") * NC + lax.axis_index("c")
    base = wid * EPW
    pltpu.sync_copy(src2_hbm.at[wid], src_v)

    def pair(j, carry):
        c0 = 2 * j
        c1 = c0 + 1
        a = pltpu.async_copy(x_hbm.at[src_v.at[c0]], xrow_a, sem_a)
        b = pltpu.async_copy(x_hbm.at[src_v.at[c1]], xrow_b, sem_b)
        a.wait()
        pltpu.sync_copy(xrow_a, xg_hbm.at[pl.ds(base + c0 * CH, CH)])
        b.wait()
        pltpu.sync_copy(xrow_b, xg_hbm.at[pl.ds(base + c1 * CH, CH)])
        return carry

    lax.fori_loop(0, PAIRS, pair, 0)
    c = NCHUNK - 1
    a = pltpu.async_copy(x_hbm.at[src_v.at[c]], xrow_a, sem_a)
    a.wait()
    pltpu.sync_copy(xrow_a, xg_hbm.at[pl.ds(base + c * CH, CH)])


@functools.lru_cache(maxsize=None)
def _sc_gather_x_call():
    return pl.kernel(
        _sc_gather_x_body,
        out_type=jax.ShapeDtypeStruct((E, XW), _F32),
        mesh=plsc.VectorSubcoreMesh(core_axis_name="c", subcore_axis_name="s",
                                    num_cores=NC, num_subcores=NS),
        scratch_types=[pltpu.VMEM((NCHUNK, CH), jnp.int32),
                       pltpu.VMEM((CH, XW), _F32),
                       pltpu.VMEM((CH, XW), _F32),
                       pltpu.SemaphoreType.DMA,
                       pltpu.SemaphoreType.DMA],
        compiler_params=pltpu.CompilerParams(use_tc_tiling_on_sc=False),
    )


def _sc_gather_q_body(q_hbm, dst2_hbm, qg_hbm,
                      dst_v, row_a, row_b, sem_a, sem_b):
    wid = lax.axis_index("s") * NC + lax.axis_index("c")
    base = wid * EPW
    pltpu.sync_copy(dst2_hbm.at[wid], dst_v)

    def pair(j, carry):
        c0 = 2 * j
        c1 = c0 + 1
        a = pltpu.async_copy(q_hbm.at[dst_v.at[c0]], row_a, sem_a)
        b = pltpu.async_copy(q_hbm.at[dst_v.at[c1]], row_b, sem_b)
        a.wait()
        pltpu.sync_copy(row_a, qg_hbm.at[pl.ds(base + c0 * CH, CH)])
        b.wait()
        pltpu.sync_copy(row_b, qg_hbm.at[pl.ds(base + c1 * CH, CH)])
        return carry

    lax.fori_loop(0, PAIRS, pair, 0)
    c = NCHUNK - 1
    a = pltpu.async_copy(q_hbm.at[dst_v.at[c]], row_a, sem_a)
    a.wait()
    pltpu.sync_copy(row_a, qg_hbm.at[pl.ds(base + c * CH, CH)])


@functools.lru_cache(maxsize=None)
def _sc_gather_q_call():
    return pl.kernel(
        _sc_gather_q_body,
        out_type=jax.ShapeDtypeStruct((E, QW), _F32),
        mesh=plsc.VectorSubcoreMesh(core_axis_name="c", subcore_axis_name="s",
                                    num_cores=NC, num_subcores=NS),
        scratch_types=[pltpu.VMEM((NCHUNK, CH), jnp.int32),
                       pltpu.VMEM((CH, QW), _F32),
                       pltpu.VMEM((CH, QW), _F32),
                       pltpu.SemaphoreType.DMA,
                       pltpu.SemaphoreType.DMA],
    )


def _edge_body(xg_ref, ea_ref, qrm_ref,
               n0w1, n0b1, n0g1, n0bb1,
               n1w1, n1b1, n1g1, n1bb1, w02, b02,
               ag1, ab1, aw, ab,
               wkv, bkv, s_ref, st_ref,
               ev_ref, ex_ref):
    qg = qrm_ref[:, 0:EMBED]
    r0 = qrm_ref[:, EMBED:EMBED + 1]
    r1 = qrm_ref[:, EMBED + 1:EMBED + 2]
    r2 = qrm_ref[:, EMBED + 2:EMBED + 3]
    r3 = qrm_ref[:, EMBED + 3:EMBED + 4]
    x0 = xg_ref[:, 0:1]
    x1 = xg_ref[:, 1:2]
    e0 = ea_ref[:, 0:1]
    e1 = ea_ref[:, 1:2]
    nx0 = x0 * r0 + x1 * r2
    nx1 = x0 * r1 + x1 * r3
    ne0 = e0 * r0 + e1 * r2
    ne1 = e0 * r1 + e1 * r3
    h0 = nx0 * n0w1[0:1, :] + nx1 * n0w1[1:2, :] + n0b1[...]
    h0 = jax.nn.relu(_ln(h0, n0g1[...], n0bb1[...]))
    h1 = ne0 * n1w1[0:1, :] + ne1 * n1w1[1:2, :] + n1b1[...]
    h1 = jax.nn.relu(_ln(h1, n1g1[...], n1bb1[...]))
    s = _dot(jnp.concatenate([h0, h1], axis=1), w02[...]) + b02[...]
    nbr = jax.nn.relu(_ln(s, ag1[...], ab1[...]))
    h = _dot(nbr, aw[...]) + ab[...]
    m = jnp.mean(h, axis=-1, keepdims=True)
    v2 = jnp.mean((h - m) ** 2, axis=-1, keepdims=True)
    nbr = (h - m) / jnp.sqrt(v2 + 1e-5)
    kv = _dot(nbr, wkv[...]) + bkv[...]
    k = kv[:, 0:EMBED]
    v = kv[:, EMBED:2 * EMBED]
    alpha = _dot(qg * k, s_ref[...])
    ex = jnp.exp(alpha)
    ev_ref[...] = v * _dot(ex, st_ref[...])
    ex_ref[...] = ex


def _edge_stage(xg, ea, qrm, smat, stmat, p):
    full = lambda shp: pl.BlockSpec(shp, lambda i: tuple(0 for _ in shp))
    row = lambda d: pl.BlockSpec((BE, d), lambda i: (i, 0))
    w = lambda k: p[k]
    vec = lambda k: p[k].reshape(1, -1)
    w02 = jnp.concatenate([p['n0_w2'], p['n1_w2']], axis=0)
    b02 = (p['n0_b2'] + p['n1_b2']).reshape(1, -1)
    wkv0 = jnp.concatenate([p['wk'], p['wv']], axis=1)
    bkv0 = jnp.concatenate([p['bk'], p['bv']])
    wkv = p['agg_g2'][:, None] * wkv0
    bkv = (p['agg_b2'] @ wkv0 + bkv0).reshape(1, -1)
    ins = [xg, ea, qrm,
           w('n0_w1'), vec('n0_b1'), vec('n0_g1'), vec('n0_bb1'),
           w('n1_w1'), vec('n1_b1'), vec('n1_g1'), vec('n1_bb1'),
           w02, b02,
           vec('agg_g1'), vec('agg_b1'), w('agg_w'), vec('agg_b'),
           wkv, bkv, smat, stmat]
    in_specs = [row(XW), row(2), row(QW)] + [
        full(a.shape) for a in ins[3:]]
    return pl.pallas_call(
        _edge_body,
        grid=(E // BE,),
        in_specs=in_specs,
        out_specs=[row(EMBED), row(HEADS)],
        out_shape=[jax.ShapeDtypeStruct((E, EMBED), _F32),
                   jax.ShapeDtypeStruct((E, HEADS), _F32)],
    )(*ins)


def _sc_scatter_ev_body(ev_hbm, dst2_hbm, z128_hbm, msgp_hbm,
                        dst_v, ev_a, ev_b, sem_a, sem_b, msg_sh):
    cid = lax.axis_index("c")
    sid = lax.axis_index("s")
    wid = sid * NC + cid
    base = wid * EPW
    pltpu.sync_copy(dst2_hbm.at[wid], dst_v)
    pltpu.sync_copy(z128_hbm.at[pl.ds(sid * STRIPE, STRIPE)],
                    msg_sh.at[pl.ds(sid * STRIPE, STRIPE)])

    @pl.when(sid == 0)
    def _():
        pltpu.sync_copy(z128_hbm.at[pl.ds(NS * STRIPE, TAIL)],
                        msg_sh.at[pl.ds(NS * STRIPE, TAIL)])

    plsc.subcore_barrier()

    def pair(j, carry):
        c0 = 2 * j
        c1 = c0 + 1
        a = pltpu.async_copy(ev_hbm.at[pl.ds(base + c0 * CH, CH)], ev_a, sem_a)
        b = pltpu.async_copy(ev_hbm.at[pl.ds(base + c1 * CH, CH)], ev_b, sem_b)
        a.wait()
        pltpu.sync_copy(ev_a, msg_sh.at[dst_v.at[c0]], add=True)
        b.wait()
        pltpu.sync_copy(ev_b, msg_sh.at[dst_v.at[c1]], add=True)
        return carry

    lax.fori_loop(0, PAIRS, pair, 0)
    c = NCHUNK - 1
    a = pltpu.async_copy(ev_hbm.at[pl.ds(base + c * CH, CH)], ev_a, sem_a)
    a.wait()
    pltpu.sync_copy(ev_a, msg_sh.at[dst_v.at[c]], add=True)
    plsc.subcore_barrier()
    pltpu.sync_copy(msg_sh.at[pl.ds(sid * STRIPE, STRIPE)],
                    msgp_hbm.at[cid, pl.ds(sid * STRIPE, STRIPE)])

    @pl.when(sid == 0)
    def _():
        pltpu.sync_copy(msg_sh.at[pl.ds(NS * STRIPE, TAIL)],
                        msgp_hbm.at[cid, pl.ds(NS * STRIPE, TAIL)])


@functools.lru_cache(maxsize=None)
def _sc_scatter_ev_call():
    return pl.kernel(
        _sc_scatter_ev_body,
        out_type=jax.ShapeDtypeStruct((NC, N, EMBED), _F32),
        mesh=plsc.VectorSubcoreMesh(core_axis_name="c", subcore_axis_name="s",
                                    num_cores=NC, num_subcores=NS),
        scratch_types=[pltpu.VMEM((NCHUNK, CH), jnp.int32),
                       pltpu.VMEM((CH, EMBED), _F32),
                       pltpu.VMEM((CH, EMBED), _F32),
                       pltpu.SemaphoreType.DMA,
                       pltpu.SemaphoreType.DMA,
                       pltpu.VMEM_SHARED((N, EMBED), _F32)],
    )


def _sc_scatter_ex_body(ex_hbm, dst2_hbm, z8_hbm, denp_hbm,
                        dst_v, ex_a, ex_b, sem_a, sem_b, den_sh):
    cid = lax.axis_index("c")
    sid = lax.axis_index("s")
    wid = sid * NC + cid
    base = wid * EPW
    pltpu.sync_copy(dst2_hbm.at[wid], dst_v)
    pltpu.sync_copy(z8_hbm.at[pl.ds(sid * NPS, NPS)],
                    den_sh.at[pl.ds(sid * NPS, NPS)])
    plsc.subcore_barrier()

    def pair(j, carry):
        c0 = 2 * j
        c1 = c0 + 1
        a = pltpu.async_copy(ex_hbm.at[pl.ds(base + c0 * CH, CH)], ex_a, sem_a)
        b = pltpu.async_copy(ex_hbm.at[pl.ds(base + c1 * CH, CH)], ex_b, sem_b)
        a.wait()
        pltpu.sync_copy(ex_a, den_sh.at[dst_v.at[c0]], add=True)
        b.wait()
        pltpu.sync_copy(ex_b, den_sh.at[dst_v.at[c1]], add=True)
        return carry

    lax.fori_loop(0, PAIRS, pair, 0)
    c = NCHUNK - 1
    a = pltpu.async_copy(ex_hbm.at[pl.ds(base + c * CH, CH)], ex_a, sem_a)
    a.wait()
    pltpu.sync_copy(ex_a, den_sh.at[dst_v.at[c]], add=True)
    plsc.subcore_barrier()
    pltpu.sync_copy(den_sh.at[pl.ds(sid * NPS, NPS)],
                    denp_hbm.at[cid, pl.ds(sid * NPS, NPS)])


@functools.lru_cache(maxsize=None)
def _sc_scatter_ex_call():
    return pl.kernel(
        _sc_scatter_ex_body,
        out_type=jax.ShapeDtypeStruct((NC, N, HEADS), _F32),
        mesh=plsc.VectorSubcoreMesh(core_axis_name="c", subcore_axis_name="s",
                                    num_cores=NC, num_subcores=NS),
        scratch_types=[pltpu.VMEM((NCHUNK, CH), jnp.int32),
                       pltpu.VMEM((CH, HEADS), _F32),
                       pltpu.VMEM((CH, HEADS), _F32),
                       pltpu.SemaphoreType.DMA,
                       pltpu.SemaphoreType.DMA,
                       pltpu.VMEM_SHARED((N, HEADS), _F32)],
        compiler_params=pltpu.CompilerParams(use_tc_tiling_on_sc=False),
    )


def _fuse_body(msgp_ref, denp_ref, c_ref, st_ref,
               wih, bih, whh, bhh, wself, bself, wout, bout,
               ln1g, ln1b, mw1, mb1, mw2, mb2, ln2g, ln2b,
               out_ref):
    den = denp_ref[0] + denp_ref[1]
    msg = (msgp_ref[0] + msgp_ref[1]) / (_dot(den, st_ref[...]) + 1e-16)
    c = c_ref[...]
    gate = jax.nn.sigmoid(_dot(msg, wih[...]) + bih[...]
                          + _dot(c, whh[...]) + bhh[...])
    o = msg + gate * (_dot(c, wself[...]) + bself[...] - msg)
    o = _dot(o, wout[...]) + bout[...]
    c1 = _ln(c + o, ln1g[...], ln1b[...])
    ff = _dot(jax.nn.relu(_dot(c1, mw1[...]) + mb1[...]), mw2[...]) + mb2[...]
    out_ref[...] = _ln(c1 + ff, ln2g[...], ln2b[...])


def _fuse_stage(msgp, denp, center, stmat, p):
    full = lambda shp: pl.BlockSpec(shp, lambda i: tuple(0 for _ in shp))
    w = lambda k: p[k]
    vec = lambda k: p[k].reshape(1, -1)
    ins = [msgp, denp, center, stmat,
           w('wih'), vec('bih'), w('whh'), vec('bhh'),
           w('wself'), vec('bself'), w('wout'), vec('bout'),
           vec('ln1_g'), vec('ln1_b'),
           w('mlp_w1'), vec('mlp_b1'), w('mlp_w2'), vec('mlp_b2'),
           vec('ln2_g'), vec('ln2_b')]
    in_specs = [pl.BlockSpec((NC, BN, EMBED), lambda i: (0, i, 0)),
                pl.BlockSpec((NC, BN, HEADS), lambda i: (0, i, 0)),
                pl.BlockSpec((BN, EMBED), lambda i: (i, 0))] + [
        full(a.shape) for a in ins[3:]]
    return pl.pallas_call(
        _fuse_body,
        grid=(N // BN,),
        in_specs=in_specs,
        out_specs=pl.BlockSpec((BN, EMBED), lambda i: (i, 0)),
        out_shape=jax.ShapeDtypeStruct((N, EMBED), _F32),
    )(*ins)


def kernel(x, edge_index, edge_attr, bos_mask, t, rotate_mat, params):
    p = params
    dst = edge_index[0].astype(jnp.int32)
    src = edge_index[1].astype(jnp.int32)
    rm4 = rotate_mat.reshape(N, 4)
    bm = bos_mask.astype(_F32).reshape(N, 1)
    bos_row = lax.dynamic_index_in_dim(p['bos'], t, axis=0, keepdims=True)
    jj = jnp.arange(EMBED)[:, None] // HEAD_DIM
    smat = (jj == jnp.arange(HEADS)[None, :]).astype(_F32)
    stmat = smat.T

    center, qs = _node_stage(x, rm4, bm, bos_row, p)
    xpad = jnp.concatenate([x, jnp.zeros((N, XW - 2), _F32)], axis=1)
    qrm_tab = jnp.concatenate([qs, rm4, jnp.zeros((N, QW - EMBED - 4), _F32)],
                              axis=1)
    src2 = src.reshape(NW, NCHUNK, CH)
    dst2 = dst.reshape(NW, NCHUNK, CH)
    xg = _sc_gather_x_call()(xpad, src2)
    qrm = _sc_gather_q_call()(qrm_tab, dst2)
    ev, ex = _edge_stage(xg, edge_attr, qrm, smat, stmat, p)
    z128 = jnp.zeros((N, EMBED), _F32)
    z8 = jnp.zeros((N, HEADS), _F32)
    msgp = _sc_scatter_ev_call()(ev, dst2, z128)
    denp = _sc_scatter_ex_call()(ex, dst2, z8)
    return _fuse_stage(msgp, denp, center, stmat, p)
